# Initial kernel scaffold; baseline (speedup 1.0000x reference)
import jax
import jax.numpy as jnp
from jax.experimental import pallas as pl


def kernel(inputs, weight):
    raise NotImplementedError("write your pallas kernel here")


# speedup vs baseline: 7.6785x; 7.6785x over previous
import functools

import jax
import jax.numpy as jnp
from jax import lax
from jax.experimental import pallas as pl
from jax.experimental.pallas import tpu as pltpu
from jax.experimental.pallas import tpu_sc as plsc

_NE = 8192
_D = 32
_CC = 0.25
_BM = 256

_NC = 2
_NS = 16
_L = 16
_NW = _NC * _NS
_CH = 128
_DP = 128
_CL = 128


def _argmin_body(a_ref, a2_ref, wt_ref, b2_ref, o_ref):
    a = a_ref[...]
    ab = jnp.dot(a, wt_ref[...])
    sq = a2_ref[...] - 2.0 * ab + b2_ref[...]
    d = jnp.sqrt(jnp.maximum(sq, 0.0))
    m = jnp.min(d, axis=1, keepdims=True)
    lane = lax.broadcasted_iota(jnp.int32, d.shape, 1)
    idx = jnp.min(jnp.where(d == m, lane, jnp.int32(2**30)), axis=1,
                  keepdims=True)
    o_ref[...] = idx


def _nearest_codes(flat, a2, wt, b2):
    n = flat.shape[0]
    grid = n // _BM
    return pl.pallas_call(
        _argmin_body,
        grid=(grid,),
        in_specs=[
            pl.BlockSpec((_BM, _D), lambda i: (i, 0)),
            pl.BlockSpec((_BM, 1), lambda i: (i, 0)),
            pl.BlockSpec((_D, _NE), lambda i: (0, 0)),
            pl.BlockSpec((1, _NE), lambda i: (0, 0)),
        ],
        out_specs=pl.BlockSpec((_BM, 1), lambda i: (i, 0)),
        out_shape=jax.ShapeDtypeStruct((n, 1), jnp.int32),
        compiler_params=pltpu.CompilerParams(
            dimension_semantics=("arbitrary",)),
    )(flat, a2, wt, b2)


@functools.cache
def _sc_gather_hist(n):
    rows_per_w = n // _CH // _NW
    hrows = n // _CH // _NS
    ne_per_s = _NE // _NS
    mesh = plsc.VectorSubcoreMesh(core_axis_name="c", subcore_axis_name="s")

    @functools.partial(
        pl.kernel,
        out_type=(
            jax.ShapeDtypeStruct((n, _DP), jnp.float32),
            jax.ShapeDtypeStruct((_NE, _CL), jnp.float32),
        ),
        mesh=mesh,
        scratch_types=[
            pltpu.VMEM((rows_per_w, _CH), jnp.int32),
            pltpu.VMEM((hrows, _CH), jnp.int32),
            pltpu.VMEM((_CH, _DP), jnp.float32),
            pltpu.VMEM((_CH, _CL), jnp.float32),
            pltpu.VMEM_SHARED((_NE, _CL), jnp.float32),
            pltpu.SemaphoreType.DMA,
        ],
    )
    def k(w_hbm, idx_hbm, q_hbm, cnt_hbm, idx_v, idx_h, rows_v, fill_v,
          cnt_sh, sem):
        core = lax.axis_index("c")
        sid = lax.axis_index("s")
        wid = sid * _NC + core

        pltpu.sync_copy(idx_hbm.at[pl.ds(wid * rows_per_w, rows_per_w)],
                        idx_v)
        for c in range(rows_per_w):
            pltpu.async_copy(w_hbm.at[idx_v.at[c]], rows_v, sem).wait()
            pltpu.sync_copy(
                rows_v, q_hbm.at[pl.ds((wid * rows_per_w + c) * _CH, _CH)])

        @pl.when(core == 0)
        def _():
            @pl.loop(0, _CH)
            def _(i):
                @pl.loop(0, _CL, step=_L)
                def _(j):
                    fill_v[i, pl.ds(j, _L)] = jnp.zeros((_L,), jnp.float32)

            @pl.loop(0, ne_per_s, step=_CH)
            def _(i):
                pltpu.sync_copy(fill_v,
                                cnt_sh.at[pl.ds(sid * ne_per_s + i, _CH)])

            pltpu.sync_copy(idx_hbm.at[pl.ds(sid * hrows, hrows)], idx_h)

            @pl.loop(0, _CH)
            def _(i):
                @pl.loop(0, _CL, step=_L)
                def _(j):
                    fill_v[i, pl.ds(j, _L)] = jnp.ones((_L,), jnp.float32)

            plsc.subcore_barrier()
            for c in range(hrows):
                pltpu.sync_copy(fill_v, cnt_sh.at[idx_h.at[c]], add=True)
            plsc.subcore_barrier()

            @pl.loop(0, ne_per_s, step=_CH)
            def _(i):
                pltpu.sync_copy(
                    cnt_sh.at[pl.ds(sid * ne_per_s + i, _CH)],
                    cnt_hbm.at[pl.ds(sid * ne_per_s + i, _CH)])

    return k


def _finish_body(x_ref, q_ref, c_ref, st_ref, loss_ref, perp_ref):
    x = x_ref[...]
    q = q_ref[...][:, :_D]
    st_ref[...] = x + (q - x)
    diff = q - x
    m = jnp.sum(diff * diff) * (1.0 / diff.size)
    loss_ref[...] = jnp.reshape(m + _CC * m, (1, 1))
    p = c_ref[...][:, 0:1] * (1.0 / x_ref.shape[0])
    ent = -jnp.sum(p * jnp.log(p + 1e-10))
    perp_ref[...] = jnp.reshape(jnp.exp(ent), (1, 1))


def _finish(flat, quant, counts):
    n = flat.shape[0]
    return pl.pallas_call(
        _finish_body,
        in_specs=[
            pl.BlockSpec((n, _D), lambda: (0, 0)),
            pl.BlockSpec((n, _DP), lambda: (0, 0)),
            pl.BlockSpec((_NE, _CL), lambda: (0, 0)),
        ],
        out_shape=(
            jax.ShapeDtypeStruct(flat.shape, jnp.float32),
            jax.ShapeDtypeStruct((1, 1), jnp.float32),
            jax.ShapeDtypeStruct((1, 1), jnp.float32),
        ),
    )(flat, quant, counts)


def kernel(inputs, weight):
    flat = inputs.reshape(-1, _D)
    n = flat.shape[0]
    a2 = jnp.sum(flat * flat, axis=1, keepdims=True)
    b2 = jnp.sum(weight * weight, axis=1)[None, :]
    wt = weight.T
    idx = _nearest_codes(flat, a2, wt, b2)
    idx_rows = idx.reshape(n // _CH, _CH)
    wpad = jnp.concatenate(
        [weight, jnp.zeros((_NE, _DP - _D), jnp.float32)], axis=1)
    quant, counts = _sc_gather_hist(n)(wpad, idx_rows)
    st, loss, perp = _finish(flat, quant, counts)
    return (loss[0, 0], st.reshape(inputs.shape), perp[0, 0], idx)
